# Initial kernel scaffold; baseline (speedup 1.0000x reference)
import jax
import jax.numpy as jnp
from jax.experimental import pallas as pl


def kernel(hidden_states, probs, routing_map):
    raise NotImplementedError("write your pallas kernel here")


# speedup vs baseline: 8.0031x; 8.0031x over previous
import jax
import jax.numpy as jnp
from jax.experimental import pallas as pl

_BT = 1024


def _body(hs_ref, p_ref, m_ref, out_ref, tpe_ref):
    m = m_ref[...]
    w = jnp.sum(p_ref[...] * m, axis=1, keepdims=True)
    out_ref[...] = hs_ref[...] * w

    @pl.when(pl.program_id(0) == 0)
    def _init():
        tpe_ref[...] = jnp.zeros_like(tpe_ref)

    tpe_ref[...] += jnp.sum(m, axis=0, keepdims=True)


def kernel(hidden_states, probs, routing_map):
    hidden_shape = hidden_states.shape
    H = hidden_shape[-1]
    T = probs.shape[0]
    E = probs.shape[1]
    hs = hidden_states.reshape(T, H)
    mask = routing_map.astype(jnp.float32)

    grid = (T // _BT,)
    out, tpe = pl.pallas_call(
        _body,
        grid=grid,
        in_specs=[
            pl.BlockSpec((_BT, H), lambda i: (i, 0)),
            pl.BlockSpec((_BT, E), lambda i: (i, 0)),
            pl.BlockSpec((_BT, E), lambda i: (i, 0)),
        ],
        out_specs=[
            pl.BlockSpec((_BT, H), lambda i: (i, 0)),
            pl.BlockSpec((1, E), lambda i: (0, 0)),
        ],
        out_shape=[
            jax.ShapeDtypeStruct((T, H), hs.dtype),
            jax.ShapeDtypeStruct((1, E), jnp.float32),
        ],
    )(hs, probs, mask)

    tokens_per_expert = tpe.reshape(E).astype(jnp.int32)
    return out.reshape(hidden_shape), tokens_per_expert
